# Initial kernel scaffold; baseline (speedup 1.0000x reference)
import jax
import jax.numpy as jnp
from jax.experimental import pallas as pl


def kernel(x, edge_index, W1, b1, W2, b2, W3, b3):
    raise NotImplementedError("write your pallas kernel here")


# speedup vs baseline: 21.3895x; 21.3895x over previous
import functools

import jax
import jax.numpy as jnp
from jax import lax
from jax.experimental import pallas as pl
from jax.experimental.pallas import tpu as pltpu
from jax.experimental.pallas import tpu_sc as plsc

F32 = jnp.float32
LANES = 128
NC = 2
NS = 16
DUMP = 32

_MESH = plsc.VectorSubcoreMesh(
    core_axis_name="c", subcore_axis_name="s", num_cores=NC, num_subcores=NS
)


IB = 16


def _run_edges(g_h, src_h, dst_h, idx_s, idx_d, r_a, r_b, s_a, s_b, acc,
               base_row, ch):
    def blk(b, carry):
        row0 = base_row + b * IB
        pltpu.sync_copy(src_h.at[pl.ds(row0, IB)], idx_s)
        pltpu.sync_copy(dst_h.at[pl.ds(row0, IB)], idx_d)
        pltpu.async_copy(g_h.at[idx_s.at[0]], r_a, s_a)

        def pair(j, c2):
            i0 = 2 * j
            i1 = i0 + 1
            i2 = jnp.where(i1 + 1 < IB, i1 + 1, 0)
            pltpu.async_copy(g_h.at[idx_s.at[i1]], r_b, s_b)
            pltpu.make_async_copy(g_h.at[idx_s.at[i0]], r_a, s_a).wait()
            pltpu.sync_copy(r_a, acc.at[idx_d.at[i0]], add=True)
            pltpu.async_copy(g_h.at[idx_s.at[i2]], r_a, s_a)
            pltpu.make_async_copy(g_h.at[idx_s.at[i1]], r_b, s_b).wait()
            pltpu.sync_copy(r_b, acc.at[idx_d.at[i1]], add=True)
            return c2

        lax.fori_loop(0, IB // 2, pair, 0)
        pltpu.make_async_copy(g_h.at[idx_s.at[0]], r_a, s_a).wait()
        return carry

    lax.fori_loop(0, ch // IB, blk, 0)


def _tile_rows_copy(s, src_ref, dst_ref, rpt, last):
    @pl.when(s < NS - 1)
    def _():
        sl = pl.ds(s * rpt, rpt)
        pltpu.sync_copy(src_ref.at[sl], dst_ref.at[sl])

    @pl.when(s == NS - 1)
    def _():
        sl = pl.ds((NS - 1) * rpt, last)
        pltpu.sync_copy(src_ref.at[sl], dst_ref.at[sl])


@functools.lru_cache(maxsize=None)
def _make_prop_feature_split(n, ch):
    rpt = -(-n // (NS * 8)) * 8
    last = n - (NS - 1) * rpt
    out_t = [jax.ShapeDtypeStruct((n, 128), F32)] * 2
    scratch = [
        pltpu.VMEM((IB, LANES), jnp.int32),
        pltpu.VMEM((IB, LANES), jnp.int32),
        pltpu.VMEM((LANES, 128), F32),
        pltpu.VMEM((LANES, 128), F32),
        pltpu.SemaphoreType.DMA,
        pltpu.SemaphoreType.DMA,
        pltpu.VMEM_SHARED((n + DUMP, 128), F32),
    ]

    @functools.partial(pl.kernel, out_type=out_t, mesh=_MESH,
                       scratch_types=scratch)
    def prop(ga_h, gb_h, src_h, dst_h, oa_h, ob_h,
             idx_s, idx_d, r_a, r_b, s_a, s_b, acc):
        c = lax.axis_index("c")
        s = lax.axis_index("s")

        @pl.when(c == 0)
        def _():
            _tile_rows_copy(s, ga_h, acc, rpt, last)

        @pl.when(c == 1)
        def _():
            _tile_rows_copy(s, gb_h, acc, rpt, last)

        plsc.subcore_barrier()

        @pl.when(c == 0)
        def _():
            _run_edges(ga_h, src_h, dst_h, idx_s, idx_d, r_a, r_b, s_a, s_b,
                       acc, s * ch, ch)

        @pl.when(c == 1)
        def _():
            _run_edges(gb_h, src_h, dst_h, idx_s, idx_d, r_a, r_b, s_a, s_b,
                       acc, s * ch, ch)

        plsc.subcore_barrier()

        @pl.when(c == 0)
        def _():
            _tile_rows_copy(s, acc, oa_h, rpt, last)

        @pl.when(c == 1)
        def _():
            _tile_rows_copy(s, acc, ob_h, rpt, last)

    return prop


@functools.lru_cache(maxsize=None)
def _make_prop_edge_split(n, ch):
    rpt = -(-n // (NS * 8)) * 8
    last = n - (NS - 1) * rpt
    out_t = [jax.ShapeDtypeStruct((n, 128), F32)] * 2
    scratch = [
        pltpu.VMEM((IB, LANES), jnp.int32),
        pltpu.VMEM((IB, LANES), jnp.int32),
        pltpu.VMEM((LANES, 128), F32),
        pltpu.VMEM((LANES, 128), F32),
        pltpu.SemaphoreType.DMA,
        pltpu.SemaphoreType.DMA,
        pltpu.VMEM_SHARED((n + DUMP, 128), F32),
    ]

    @functools.partial(pl.kernel, out_type=out_t, mesh=_MESH,
                       scratch_types=scratch)
    def prop(g_h, z_h, src_h, dst_h, oa_h, ob_h,
             idx_s, idx_d, r_a, r_b, s_a, s_b, acc):
        c = lax.axis_index("c")
        s = lax.axis_index("s")
        wid = s * NC + c

        @pl.when(c == 0)
        def _():
            _tile_rows_copy(s, g_h, acc, rpt, last)

        @pl.when(c == 1)
        def _():
            _tile_rows_copy(s, z_h, acc, rpt, last)

        plsc.subcore_barrier()
        _run_edges(g_h, src_h, dst_h, idx_s, idx_d, r_a, r_b, s_a, s_b,
                   acc, wid * ch, ch)
        plsc.subcore_barrier()

        @pl.when(c == 0)
        def _():
            _tile_rows_copy(s, acc, oa_h, rpt, last)

        @pl.when(c == 1)
        def _():
            _tile_rows_copy(s, acc, ob_h, rpt, last)

    return prop


@functools.lru_cache(maxsize=None)
def _make_deg(n_acc, deg_tile, ch):
    out_t = [jax.ShapeDtypeStruct((n_acc,), F32)] * 2
    scratch = [
        pltpu.VMEM((ch, LANES), jnp.int32),
        pltpu.VMEM((LANES,), F32),
        pltpu.VMEM((deg_tile,), F32),
        pltpu.VMEM_SHARED((n_acc,), F32),
    ]

    @functools.partial(pl.kernel, out_type=out_t, mesh=_MESH,
                       scratch_types=scratch)
    def deg(dst_h, oa_h, ob_h, idxbuf, ones_v, zbuf, dacc):
        c = lax.axis_index("c")
        s = lax.axis_index("s")
        wid = s * NC + c

        def fill_z(i, carry):
            zbuf[pl.ds(i * 16, 16)] = jnp.zeros((16,), F32)
            return carry

        lax.fori_loop(0, deg_tile // 16, fill_z, 0)

        def fill_o(i, carry):
            ones_v[pl.ds(i * 16, 16)] = jnp.full((16,), 1.0, F32)
            return carry

        lax.fori_loop(0, LANES // 16, fill_o, 0)

        sl = pl.ds(s * deg_tile, deg_tile)
        pltpu.sync_copy(zbuf, dacc.at[sl])
        plsc.subcore_barrier()

        pltpu.sync_copy(dst_h.at[pl.ds(wid * ch, ch)], idxbuf)

        def st(i, carry):
            pltpu.sync_copy(ones_v, dacc.at[idxbuf.at[i]], add=True)
            return carry

        lax.fori_loop(0, ch, st, 0)
        plsc.subcore_barrier()

        @pl.when(c == 0)
        def _():
            pltpu.sync_copy(dacc.at[sl], oa_h.at[sl])

        @pl.when(c == 1)
        def _():
            pltpu.sync_copy(dacc.at[sl], ob_h.at[sl])

    return deg


_ROWS = 1000


def _row_spec(w):
    return pl.BlockSpec((_ROWS, w), lambda i: (i, 0))


def _full_spec(h, w):
    return pl.BlockSpec((h, w), lambda i: (0, 0))


@functools.lru_cache(maxsize=None)
def _make_k1(n, fin, fout):
    grid = (n // _ROWS,)

    def body(x_r, w_r, da_r, db_r, ga_r, gb_r, dv_r):
        dinv = lax.rsqrt(da_r[...] + db_r[...] + 1.0)
        g = jnp.dot(x_r[...], w_r[...], preferred_element_type=F32) * dinv
        ga_r[...] = g[:, : fout // 2]
        gb_r[...] = g[:, fout // 2:]
        dv_r[...] = dinv

    return pl.pallas_call(
        body,
        grid=grid,
        in_specs=[_row_spec(fin), _full_spec(fin, fout),
                  _row_spec(1), _row_spec(1)],
        out_specs=[_row_spec(fout // 2), _row_spec(fout // 2), _row_spec(1)],
        out_shape=[jax.ShapeDtypeStruct((n, fout // 2), F32),
                   jax.ShapeDtypeStruct((n, fout // 2), F32),
                   jax.ShapeDtypeStruct((n, 1), F32)],
    )


@functools.lru_cache(maxsize=None)
def _make_mid(n, fin, fout, split):
    grid = (n // _ROWS,)

    def body(pa_r, pb_r, dv_r, b_r, w_r, *outs):
        dinv = dv_r[...]
        p = jnp.concatenate([pa_r[...], pb_r[...]], axis=1)
        a = jnp.maximum(p * dinv + b_r[...], 0.0)
        g = jnp.dot(a, w_r[...], preferred_element_type=F32) * dinv
        if split:
            outs[0][...] = g[:, : fout // 2]
            outs[1][...] = g[:, fout // 2:]
        else:
            outs[0][...] = g

    if split:
        out_specs = [_row_spec(fout // 2), _row_spec(fout // 2)]
        out_shape = [jax.ShapeDtypeStruct((n, fout // 2), F32)] * 2
    else:
        out_specs = [_row_spec(fout)]
        out_shape = [jax.ShapeDtypeStruct((n, fout), F32)]

    return pl.pallas_call(
        body,
        grid=grid,
        in_specs=[_row_spec(fin // 2), _row_spec(fin // 2), _row_spec(1),
                  _full_spec(1, fin), _full_spec(fin, fout)],
        out_specs=out_specs,
        out_shape=out_shape,
    )


@functools.lru_cache(maxsize=None)
def _make_k4(n, f):
    grid = (n // _ROWS,)

    def body(pa_r, pb_r, dv_r, b_r, o_r):
        o_r[...] = jax.nn.sigmoid(
            (pa_r[...] + pb_r[...]) * dv_r[...] + b_r[...])

    return pl.pallas_call(
        body,
        grid=grid,
        in_specs=[_row_spec(f), _row_spec(f), _row_spec(1), _full_spec(1, f)],
        out_specs=_row_spec(f),
        out_shape=jax.ShapeDtypeStruct((n, f), F32),
    )


def kernel(x, edge_index, W1, b1, W2, b2, W3, b3):
    n = x.shape[0]
    e = edge_index.shape[1]

    src = edge_index[0].astype(jnp.int32)
    dst = edge_index[1].astype(jnp.int32)

    ch3 = -(-e // (NC * NS * LANES))
    ch3 += ch3 % 2
    ch1 = 2 * ch3
    e_pad = NC * NS * ch3 * LANES
    npad = e_pad - e
    ar = jnp.arange(npad, dtype=jnp.int32)
    src2 = jnp.concatenate([src, ar % n]).reshape(-1, LANES)
    dst2 = jnp.concatenate([dst, n + (ar % DUMP)]).reshape(-1, LANES)

    deg_tile = -(-(n + DUMP) // (NS * 16)) * 16
    n_acc = NS * deg_tile
    dega, degb = _make_deg(n_acc, deg_tile, ch3)(dst2)
    da = dega[:, None]
    db = degb[:, None]

    prop = _make_prop_feature_split(n, ch1)
    zeros_tab = jnp.zeros((n, 128), F32)

    g1a, g1b, dinv = _make_k1(n, 128, 256)(x, W1, da, db)
    p1a, p1b = prop(g1a, g1b, src2, dst2)
    g2a, g2b = _make_mid(n, 256, 256, True)(
        p1a, p1b, dinv, b1.reshape(1, -1), W2)
    p2a, p2b = prop(g2a, g2b, src2, dst2)
    (g3,) = _make_mid(n, 256, 128, False)(
        p2a, p2b, dinv, b2.reshape(1, -1), W3)
    p3a, p3b = _make_prop_edge_split(n, ch3)(g3, zeros_tab, src2, dst2)
    out = _make_k4(n, 128)(p3a, p3b, dinv, b3.reshape(1, -1))
    return out


# speedup vs baseline: 23.2070x; 1.0850x over previous
import functools

import jax
import jax.numpy as jnp
from jax import lax
from jax.experimental import pallas as pl
from jax.experimental.pallas import tpu as pltpu
from jax.experimental.pallas import tpu_sc as plsc

F32 = jnp.float32
LANES = 128
NC = 2
NS = 16
DUMP = 32

_MESH = plsc.VectorSubcoreMesh(
    core_axis_name="c", subcore_axis_name="s", num_cores=NC, num_subcores=NS
)


IB = 40


def _run_edges(g_h, src_h, dst_h, idx_s, idx_d, r_a, r_b, s_a, s_b, acc,
               base_row, ch):
    def blk(b, carry):
        row0 = base_row + b * IB
        pltpu.sync_copy(src_h.at[pl.ds(row0, IB)], idx_s)
        pltpu.sync_copy(dst_h.at[pl.ds(row0, IB)], idx_d)
        pltpu.async_copy(g_h.at[idx_s.at[0]], r_a, s_a)

        def pair(j, c2):
            i0 = 2 * j
            i1 = i0 + 1
            i2 = jnp.where(i1 + 1 < IB, i1 + 1, 0)
            pltpu.async_copy(g_h.at[idx_s.at[i1]], r_b, s_b)
            pltpu.make_async_copy(g_h.at[idx_s.at[i0]], r_a, s_a).wait()
            pltpu.sync_copy(r_a, acc.at[idx_d.at[i0]], add=True)
            pltpu.async_copy(g_h.at[idx_s.at[i2]], r_a, s_a)
            pltpu.make_async_copy(g_h.at[idx_s.at[i1]], r_b, s_b).wait()
            pltpu.sync_copy(r_b, acc.at[idx_d.at[i1]], add=True)
            return c2

        lax.fori_loop(0, IB // 2, pair, 0)
        pltpu.make_async_copy(g_h.at[idx_s.at[0]], r_a, s_a).wait()
        return carry

    lax.fori_loop(0, ch // IB, blk, 0)


def _tile_rows_copy(s, src_ref, dst_ref, rpt, last):
    @pl.when(s < NS - 1)
    def _():
        sl = pl.ds(s * rpt, rpt)
        pltpu.sync_copy(src_ref.at[sl], dst_ref.at[sl])

    @pl.when(s == NS - 1)
    def _():
        sl = pl.ds((NS - 1) * rpt, last)
        pltpu.sync_copy(src_ref.at[sl], dst_ref.at[sl])


@functools.lru_cache(maxsize=None)
def _make_prop_feature_split(n, ch):
    rpt = -(-n // (NS * 8)) * 8
    last = n - (NS - 1) * rpt
    out_t = [jax.ShapeDtypeStruct((n, 128), F32)] * 2
    scratch = [
        pltpu.VMEM((IB, LANES), jnp.int32),
        pltpu.VMEM((IB, LANES), jnp.int32),
        pltpu.VMEM((LANES, 128), F32),
        pltpu.VMEM((LANES, 128), F32),
        pltpu.SemaphoreType.DMA,
        pltpu.SemaphoreType.DMA,
        pltpu.VMEM_SHARED((n + DUMP, 128), F32),
    ]

    @functools.partial(pl.kernel, out_type=out_t, mesh=_MESH,
                       scratch_types=scratch)
    def prop(ga_h, gb_h, src_h, dst_h, oa_h, ob_h,
             idx_s, idx_d, r_a, r_b, s_a, s_b, acc):
        c = lax.axis_index("c")
        s = lax.axis_index("s")

        @pl.when(c == 0)
        def _():
            _tile_rows_copy(s, ga_h, acc, rpt, last)

        @pl.when(c == 1)
        def _():
            _tile_rows_copy(s, gb_h, acc, rpt, last)

        plsc.subcore_barrier()

        @pl.when(c == 0)
        def _():
            _run_edges(ga_h, src_h, dst_h, idx_s, idx_d, r_a, r_b, s_a, s_b,
                       acc, s * ch, ch)

        @pl.when(c == 1)
        def _():
            _run_edges(gb_h, src_h, dst_h, idx_s, idx_d, r_a, r_b, s_a, s_b,
                       acc, s * ch, ch)

        plsc.subcore_barrier()

        @pl.when(c == 0)
        def _():
            _tile_rows_copy(s, acc, oa_h, rpt, last)

        @pl.when(c == 1)
        def _():
            _tile_rows_copy(s, acc, ob_h, rpt, last)

    return prop


@functools.lru_cache(maxsize=None)
def _make_prop_edge_split(n, ch):
    rpt = -(-n // (NS * 8)) * 8
    last = n - (NS - 1) * rpt
    out_t = [jax.ShapeDtypeStruct((n, 128), F32)] * 2
    scratch = [
        pltpu.VMEM((IB, LANES), jnp.int32),
        pltpu.VMEM((IB, LANES), jnp.int32),
        pltpu.VMEM((LANES, 128), F32),
        pltpu.VMEM((LANES, 128), F32),
        pltpu.SemaphoreType.DMA,
        pltpu.SemaphoreType.DMA,
        pltpu.VMEM_SHARED((n + DUMP, 128), F32),
    ]

    @functools.partial(pl.kernel, out_type=out_t, mesh=_MESH,
                       scratch_types=scratch)
    def prop(g_h, z_h, src_h, dst_h, oa_h, ob_h,
             idx_s, idx_d, r_a, r_b, s_a, s_b, acc):
        c = lax.axis_index("c")
        s = lax.axis_index("s")
        wid = s * NC + c

        @pl.when(c == 0)
        def _():
            _tile_rows_copy(s, g_h, acc, rpt, last)

        @pl.when(c == 1)
        def _():
            _tile_rows_copy(s, z_h, acc, rpt, last)

        plsc.subcore_barrier()
        _run_edges(g_h, src_h, dst_h, idx_s, idx_d, r_a, r_b, s_a, s_b,
                   acc, wid * ch, ch)
        plsc.subcore_barrier()

        @pl.when(c == 0)
        def _():
            _tile_rows_copy(s, acc, oa_h, rpt, last)

        @pl.when(c == 1)
        def _():
            _tile_rows_copy(s, acc, ob_h, rpt, last)

    return prop


@functools.lru_cache(maxsize=None)
def _make_deg(n_acc, deg_tile, ch):
    out_t = [jax.ShapeDtypeStruct((n_acc,), F32)] * 2
    scratch = [
        pltpu.VMEM((ch, LANES), jnp.int32),
        pltpu.VMEM((LANES,), F32),
        pltpu.VMEM((deg_tile,), F32),
        pltpu.VMEM_SHARED((n_acc,), F32),
        pltpu.SemaphoreType.DMA,
    ]

    @functools.partial(pl.kernel, out_type=out_t, mesh=_MESH,
                       scratch_types=scratch)
    def deg(dst_h, oa_h, ob_h, idxbuf, ones_v, zbuf, dacc, sem):
        c = lax.axis_index("c")
        s = lax.axis_index("s")
        wid = s * NC + c

        def fill_z(i, carry):
            zbuf[pl.ds(i * 16, 16)] = jnp.zeros((16,), F32)
            return carry

        lax.fori_loop(0, deg_tile // 16, fill_z, 0)

        def fill_o(i, carry):
            ones_v[pl.ds(i * 16, 16)] = jnp.full((16,), 1.0, F32)
            return carry

        lax.fori_loop(0, LANES // 16, fill_o, 0)

        sl = pl.ds(s * deg_tile, deg_tile)
        pltpu.sync_copy(zbuf, dacc.at[sl])
        plsc.subcore_barrier()

        pltpu.sync_copy(dst_h.at[pl.ds(wid * ch, ch)], idxbuf)

        def st(i, carry):
            pltpu.async_copy(ones_v, dacc.at[idxbuf.at[i]], sem, add=True)
            return carry

        lax.fori_loop(0, ch, st, 0)

        def dr(i, carry):
            pltpu.make_async_copy(ones_v, dacc.at[idxbuf.at[i]], sem).wait()
            return carry

        lax.fori_loop(0, ch, dr, 0)
        plsc.subcore_barrier()

        @pl.when(c == 0)
        def _():
            pltpu.sync_copy(dacc.at[sl], oa_h.at[sl])

        @pl.when(c == 1)
        def _():
            pltpu.sync_copy(dacc.at[sl], ob_h.at[sl])

    return deg


_ROWS = 1000


def _row_spec(w):
    return pl.BlockSpec((_ROWS, w), lambda i: (i, 0))


def _full_spec(h, w):
    return pl.BlockSpec((h, w), lambda i: (0, 0))


@functools.lru_cache(maxsize=None)
def _make_k1(n, fin, fout):
    grid = (n // _ROWS,)

    def body(x_r, w_r, da_r, db_r, ga_r, gb_r, dv_r):
        dinv = lax.rsqrt(da_r[...] + db_r[...] + 1.0)
        g = jnp.dot(x_r[...], w_r[...], preferred_element_type=F32) * dinv
        ga_r[...] = g[:, : fout // 2]
        gb_r[...] = g[:, fout // 2:]
        dv_r[...] = dinv

    return pl.pallas_call(
        body,
        grid=grid,
        in_specs=[_row_spec(fin), _full_spec(fin, fout),
                  _row_spec(1), _row_spec(1)],
        out_specs=[_row_spec(fout // 2), _row_spec(fout // 2), _row_spec(1)],
        out_shape=[jax.ShapeDtypeStruct((n, fout // 2), F32),
                   jax.ShapeDtypeStruct((n, fout // 2), F32),
                   jax.ShapeDtypeStruct((n, 1), F32)],
    )


@functools.lru_cache(maxsize=None)
def _make_mid(n, fin, fout, split):
    grid = (n // _ROWS,)

    def body(pa_r, pb_r, dv_r, b_r, w_r, *outs):
        dinv = dv_r[...]
        p = jnp.concatenate([pa_r[...], pb_r[...]], axis=1)
        a = jnp.maximum(p * dinv + b_r[...], 0.0)
        g = jnp.dot(a, w_r[...], preferred_element_type=F32) * dinv
        if split:
            outs[0][...] = g[:, : fout // 2]
            outs[1][...] = g[:, fout // 2:]
        else:
            outs[0][...] = g

    if split:
        out_specs = [_row_spec(fout // 2), _row_spec(fout // 2)]
        out_shape = [jax.ShapeDtypeStruct((n, fout // 2), F32)] * 2
    else:
        out_specs = [_row_spec(fout)]
        out_shape = [jax.ShapeDtypeStruct((n, fout), F32)]

    return pl.pallas_call(
        body,
        grid=grid,
        in_specs=[_row_spec(fin // 2), _row_spec(fin // 2), _row_spec(1),
                  _full_spec(1, fin), _full_spec(fin, fout)],
        out_specs=out_specs,
        out_shape=out_shape,
    )


@functools.lru_cache(maxsize=None)
def _make_k4(n, f):
    grid = (n // _ROWS,)

    def body(pa_r, pb_r, dv_r, b_r, o_r):
        o_r[...] = jax.nn.sigmoid(
            (pa_r[...] + pb_r[...]) * dv_r[...] + b_r[...])

    return pl.pallas_call(
        body,
        grid=grid,
        in_specs=[_row_spec(f), _row_spec(f), _row_spec(1), _full_spec(1, f)],
        out_specs=_row_spec(f),
        out_shape=jax.ShapeDtypeStruct((n, f), F32),
    )


def kernel(x, edge_index, W1, b1, W2, b2, W3, b3):
    n = x.shape[0]
    e = edge_index.shape[1]

    src = edge_index[0].astype(jnp.int32)
    dst = edge_index[1].astype(jnp.int32)

    ch3 = -(-e // (NC * NS * LANES))
    ch3 += ch3 % 2
    ch1 = 2 * ch3
    e_pad = NC * NS * ch3 * LANES
    npad = e_pad - e
    ar = jnp.arange(npad, dtype=jnp.int32)
    src2 = jnp.concatenate([src, ar % n]).reshape(-1, LANES)
    dst2 = jnp.concatenate([dst, n + (ar % DUMP)]).reshape(-1, LANES)

    deg_tile = -(-(n + DUMP) // (NS * 16)) * 16
    n_acc = NS * deg_tile
    dega, degb = _make_deg(n_acc, deg_tile, ch3)(dst2)
    da = dega[:, None]
    db = degb[:, None]

    prop = _make_prop_feature_split(n, ch1)
    zeros_tab = jnp.zeros((n, 128), F32)

    g1a, g1b, dinv = _make_k1(n, 128, 256)(x, W1, da, db)
    p1a, p1b = prop(g1a, g1b, src2, dst2)
    g2a, g2b = _make_mid(n, 256, 256, True)(
        p1a, p1b, dinv, b1.reshape(1, -1), W2)
    p2a, p2b = prop(g2a, g2b, src2, dst2)
    (g3,) = _make_mid(n, 256, 128, False)(
        p2a, p2b, dinv, b2.reshape(1, -1), W3)
    p3a, p3b = _make_prop_edge_split(n, ch3)(g3, zeros_tab, src2, dst2)
    out = _make_k4(n, 128)(p3a, p3b, dinv, b3.reshape(1, -1))
    return out


# speedup vs baseline: 23.7266x; 1.0224x over previous
import functools

import jax
import jax.numpy as jnp
from jax import lax
from jax.experimental import pallas as pl
from jax.experimental.pallas import tpu as pltpu
from jax.experimental.pallas import tpu_sc as plsc

F32 = jnp.float32
LANES = 128
NC = 2
NS = 16
DUMP = 32

_MESH = plsc.VectorSubcoreMesh(
    core_axis_name="c", subcore_axis_name="s", num_cores=NC, num_subcores=NS
)


IB = 8


def _run_edges(g_h, src_h, dst_h, ixs, ixd, jxs, jxd, r_a, r_b,
               s_a, s_b, s_i, acc, base_row, ch):
    nblk = ch // IB

    def pair(cs, cd, i0):
        i1 = i0 + 1
        pltpu.async_copy(g_h.at[cs.at[i1]], r_b, s_b)
        pltpu.make_async_copy(g_h.at[cs.at[i0]], r_a, s_a).wait()
        pltpu.sync_copy(r_a, acc.at[cd.at[i0]], add=True)
        pltpu.async_copy(g_h.at[cs.at[i0 + 2]], r_a, s_a)
        pltpu.make_async_copy(g_h.at[cs.at[i1]], r_b, s_b).wait()
        pltpu.sync_copy(r_b, acc.at[cd.at[i1]], add=True)

    def block(b, cs, cd, ns, nd):
        nb = jnp.where(b + 1 < nblk, b + 1, 0)
        row_n = base_row + nb * IB
        pltpu.async_copy(src_h.at[pl.ds(row_n, IB)], ns, s_i)
        pltpu.async_copy(dst_h.at[pl.ds(row_n, IB)], nd, s_i)

        def mid(j, c2):
            pair(cs, cd, 2 * j)
            return c2

        lax.fori_loop(0, IB // 2 - 1, mid, 0)
        pltpu.make_async_copy(src_h.at[pl.ds(row_n, IB)], ns, s_i).wait()
        pltpu.make_async_copy(dst_h.at[pl.ds(row_n, IB)], nd, s_i).wait()
        i0 = IB - 2
        i1 = IB - 1
        pltpu.async_copy(g_h.at[cs.at[i1]], r_b, s_b)
        pltpu.make_async_copy(g_h.at[cs.at[i0]], r_a, s_a).wait()
        pltpu.sync_copy(r_a, acc.at[cd.at[i0]], add=True)
        pltpu.async_copy(g_h.at[ns.at[0]], r_a, s_a)
        pltpu.make_async_copy(g_h.at[cs.at[i1]], r_b, s_b).wait()
        pltpu.sync_copy(r_b, acc.at[cd.at[i1]], add=True)

    pltpu.sync_copy(src_h.at[pl.ds(base_row, IB)], ixs)
    pltpu.sync_copy(dst_h.at[pl.ds(base_row, IB)], ixd)
    pltpu.async_copy(g_h.at[ixs.at[0]], r_a, s_a)

    def two(t, c2):
        block(2 * t, ixs, ixd, jxs, jxd)
        block(2 * t + 1, jxs, jxd, ixs, ixd)
        return c2

    lax.fori_loop(0, nblk // 2, two, 0)
    pltpu.make_async_copy(g_h.at[ixs.at[0]], r_a, s_a).wait()


def _tile_rows_copy(s, src_ref, dst_ref, rpt, last):
    @pl.when(s < NS - 1)
    def _():
        sl = pl.ds(s * rpt, rpt)
        pltpu.sync_copy(src_ref.at[sl], dst_ref.at[sl])

    @pl.when(s == NS - 1)
    def _():
        sl = pl.ds((NS - 1) * rpt, last)
        pltpu.sync_copy(src_ref.at[sl], dst_ref.at[sl])


@functools.lru_cache(maxsize=None)
def _make_prop_feature_split(n, ch):
    rpt = -(-n // (NS * 8)) * 8
    last = n - (NS - 1) * rpt
    out_t = [jax.ShapeDtypeStruct((n, 128), F32)] * 2
    scratch = [
        pltpu.VMEM((IB, LANES), jnp.int32),
        pltpu.VMEM((IB, LANES), jnp.int32),
        pltpu.VMEM((IB, LANES), jnp.int32),
        pltpu.VMEM((IB, LANES), jnp.int32),
        pltpu.VMEM((LANES, 128), F32),
        pltpu.VMEM((LANES, 128), F32),
        pltpu.SemaphoreType.DMA,
        pltpu.SemaphoreType.DMA,
        pltpu.SemaphoreType.DMA,
        pltpu.VMEM_SHARED((n + DUMP, 128), F32),
    ]

    @functools.partial(pl.kernel, out_type=out_t, mesh=_MESH,
                       scratch_types=scratch)
    def prop(ga_h, gb_h, src_h, dst_h, oa_h, ob_h,
             ixs, ixd, jxs, jxd, r_a, r_b, s_a, s_b, s_i, acc):
        c = lax.axis_index("c")
        s = lax.axis_index("s")

        @pl.when(c == 0)
        def _():
            _tile_rows_copy(s, ga_h, acc, rpt, last)

        @pl.when(c == 1)
        def _():
            _tile_rows_copy(s, gb_h, acc, rpt, last)

        plsc.subcore_barrier()

        @pl.when(c == 0)
        def _():
            _run_edges(ga_h, src_h, dst_h, ixs, ixd, jxs, jxd, r_a, r_b,
                       s_a, s_b, s_i, acc, s * ch, ch)

        @pl.when(c == 1)
        def _():
            _run_edges(gb_h, src_h, dst_h, ixs, ixd, jxs, jxd, r_a, r_b,
                       s_a, s_b, s_i, acc, s * ch, ch)

        plsc.subcore_barrier()

        @pl.when(c == 0)
        def _():
            _tile_rows_copy(s, acc, oa_h, rpt, last)

        @pl.when(c == 1)
        def _():
            _tile_rows_copy(s, acc, ob_h, rpt, last)

    return prop


@functools.lru_cache(maxsize=None)
def _make_prop_edge_split(n, ch):
    rpt = -(-n // (NS * 8)) * 8
    last = n - (NS - 1) * rpt
    out_t = [jax.ShapeDtypeStruct((n, 128), F32)] * 2
    scratch = [
        pltpu.VMEM((IB, LANES), jnp.int32),
        pltpu.VMEM((IB, LANES), jnp.int32),
        pltpu.VMEM((IB, LANES), jnp.int32),
        pltpu.VMEM((IB, LANES), jnp.int32),
        pltpu.VMEM((LANES, 128), F32),
        pltpu.VMEM((LANES, 128), F32),
        pltpu.SemaphoreType.DMA,
        pltpu.SemaphoreType.DMA,
        pltpu.SemaphoreType.DMA,
        pltpu.VMEM_SHARED((n + DUMP, 128), F32),
    ]

    @functools.partial(pl.kernel, out_type=out_t, mesh=_MESH,
                       scratch_types=scratch)
    def prop(g_h, z_h, src_h, dst_h, oa_h, ob_h,
             ixs, ixd, jxs, jxd, r_a, r_b, s_a, s_b, s_i, acc):
        c = lax.axis_index("c")
        s = lax.axis_index("s")
        wid = s * NC + c

        @pl.when(c == 0)
        def _():
            _tile_rows_copy(s, g_h, acc, rpt, last)

        @pl.when(c == 1)
        def _():
            _tile_rows_copy(s, z_h, acc, rpt, last)

        plsc.subcore_barrier()
        _run_edges(g_h, src_h, dst_h, ixs, ixd, jxs, jxd, r_a, r_b,
                   s_a, s_b, s_i, acc, wid * ch, ch)
        plsc.subcore_barrier()

        @pl.when(c == 0)
        def _():
            _tile_rows_copy(s, acc, oa_h, rpt, last)

        @pl.when(c == 1)
        def _():
            _tile_rows_copy(s, acc, ob_h, rpt, last)

    return prop


@functools.lru_cache(maxsize=None)
def _make_deg(n_acc, deg_tile, ch):
    out_t = [jax.ShapeDtypeStruct((n_acc,), F32)] * 2
    scratch = [
        pltpu.VMEM((ch, LANES), jnp.int32),
        pltpu.VMEM((LANES,), F32),
        pltpu.VMEM((deg_tile,), F32),
        pltpu.VMEM_SHARED((n_acc,), F32),
        pltpu.SemaphoreType.DMA,
    ]

    @functools.partial(pl.kernel, out_type=out_t, mesh=_MESH,
                       scratch_types=scratch)
    def deg(dst_h, oa_h, ob_h, idxbuf, ones_v, zbuf, dacc, sem):
        c = lax.axis_index("c")
        s = lax.axis_index("s")
        wid = s * NC + c

        def fill_z(i, carry):
            zbuf[pl.ds(i * 16, 16)] = jnp.zeros((16,), F32)
            return carry

        lax.fori_loop(0, deg_tile // 16, fill_z, 0)

        def fill_o(i, carry):
            ones_v[pl.ds(i * 16, 16)] = jnp.full((16,), 1.0, F32)
            return carry

        lax.fori_loop(0, LANES // 16, fill_o, 0)

        sl = pl.ds(s * deg_tile, deg_tile)
        pltpu.sync_copy(zbuf, dacc.at[sl])
        plsc.subcore_barrier()

        pltpu.sync_copy(dst_h.at[pl.ds(wid * ch, ch)], idxbuf)

        def st(i, carry):
            pltpu.async_copy(ones_v, dacc.at[idxbuf.at[i]], sem, add=True)
            return carry

        lax.fori_loop(0, ch, st, 0)

        def dr(i, carry):
            pltpu.make_async_copy(ones_v, dacc.at[idxbuf.at[i]], sem).wait()
            return carry

        lax.fori_loop(0, ch, dr, 0)
        plsc.subcore_barrier()

        @pl.when(c == 0)
        def _():
            pltpu.sync_copy(dacc.at[sl], oa_h.at[sl])

        @pl.when(c == 1)
        def _():
            pltpu.sync_copy(dacc.at[sl], ob_h.at[sl])

    return deg


_ROWS = 1000


def _row_spec(w):
    return pl.BlockSpec((_ROWS, w), lambda i: (i, 0))


def _full_spec(h, w):
    return pl.BlockSpec((h, w), lambda i: (0, 0))


@functools.lru_cache(maxsize=None)
def _make_k1a(n, fin, fout):
    grid = (n // _ROWS,)

    def body(x_r, w_r, ha_r, hb_r):
        h = jnp.dot(x_r[...], w_r[...], preferred_element_type=F32)
        ha_r[...] = h[:, : fout // 2]
        hb_r[...] = h[:, fout // 2:]

    return pl.pallas_call(
        body,
        grid=grid,
        in_specs=[_row_spec(fin), _full_spec(fin, fout)],
        out_specs=[_row_spec(fout // 2), _row_spec(fout // 2)],
        out_shape=[jax.ShapeDtypeStruct((n, fout // 2), F32),
                   jax.ShapeDtypeStruct((n, fout // 2), F32)],
    )


@functools.lru_cache(maxsize=None)
def _make_k1b(n, half):
    grid = (n // _ROWS,)

    def body(ha_r, hb_r, da_r, db_r, ga_r, gb_r, dv_r):
        dinv = lax.rsqrt(da_r[...] + db_r[...] + 1.0)
        ga_r[...] = ha_r[...] * dinv
        gb_r[...] = hb_r[...] * dinv
        dv_r[...] = dinv

    return pl.pallas_call(
        body,
        grid=grid,
        in_specs=[_row_spec(half), _row_spec(half),
                  _row_spec(1), _row_spec(1)],
        out_specs=[_row_spec(half), _row_spec(half), _row_spec(1)],
        out_shape=[jax.ShapeDtypeStruct((n, half), F32),
                   jax.ShapeDtypeStruct((n, half), F32),
                   jax.ShapeDtypeStruct((n, 1), F32)],
    )


@functools.lru_cache(maxsize=None)
def _make_mid(n, fin, fout, split):
    grid = (n // _ROWS,)

    def body(pa_r, pb_r, dv_r, b_r, w_r, *outs):
        dinv = dv_r[...]
        p = jnp.concatenate([pa_r[...], pb_r[...]], axis=1)
        a = jnp.maximum(p * dinv + b_r[...], 0.0)
        g = jnp.dot(a, w_r[...], preferred_element_type=F32) * dinv
        if split:
            outs[0][...] = g[:, : fout // 2]
            outs[1][...] = g[:, fout // 2:]
        else:
            outs[0][...] = g

    if split:
        out_specs = [_row_spec(fout // 2), _row_spec(fout // 2)]
        out_shape = [jax.ShapeDtypeStruct((n, fout // 2), F32)] * 2
    else:
        out_specs = [_row_spec(fout)]
        out_shape = [jax.ShapeDtypeStruct((n, fout), F32)]

    return pl.pallas_call(
        body,
        grid=grid,
        in_specs=[_row_spec(fin // 2), _row_spec(fin // 2), _row_spec(1),
                  _full_spec(1, fin), _full_spec(fin, fout)],
        out_specs=out_specs,
        out_shape=out_shape,
    )


@functools.lru_cache(maxsize=None)
def _make_k4(n, f):
    grid = (n // _ROWS,)

    def body(pa_r, pb_r, dv_r, b_r, o_r):
        o_r[...] = jax.nn.sigmoid(
            (pa_r[...] + pb_r[...]) * dv_r[...] + b_r[...])

    return pl.pallas_call(
        body,
        grid=grid,
        in_specs=[_row_spec(f), _row_spec(f), _row_spec(1), _full_spec(1, f)],
        out_specs=_row_spec(f),
        out_shape=jax.ShapeDtypeStruct((n, f), F32),
    )


def kernel(x, edge_index, W1, b1, W2, b2, W3, b3):
    n = x.shape[0]
    e = edge_index.shape[1]

    src = edge_index[0].astype(jnp.int32)
    dst = edge_index[1].astype(jnp.int32)

    ch3 = -(-e // (NC * NS * LANES))
    ch3 += ch3 % 2
    ch1 = 2 * ch3
    e_pad = NC * NS * ch3 * LANES
    npad = e_pad - e
    ar = jnp.arange(npad, dtype=jnp.int32)
    src2 = jnp.concatenate([src, ar % n]).reshape(-1, LANES)
    dst2 = jnp.concatenate([dst, n + (ar % DUMP)]).reshape(-1, LANES)

    deg_tile = -(-(n + DUMP) // (NS * 16)) * 16
    n_acc = NS * deg_tile
    dega, degb = _make_deg(n_acc, deg_tile, ch3)(dst2)
    da = dega[:, None]
    db = degb[:, None]

    prop = _make_prop_feature_split(n, ch1)
    zeros_tab = jnp.zeros((n, 128), F32)

    h1a, h1b = _make_k1a(n, 128, 256)(x, W1)
    g1a, g1b, dinv = _make_k1b(n, 128)(h1a, h1b, da, db)
    p1a, p1b = prop(g1a, g1b, src2, dst2)
    g2a, g2b = _make_mid(n, 256, 256, True)(
        p1a, p1b, dinv, b1.reshape(1, -1), W2)
    p2a, p2b = prop(g2a, g2b, src2, dst2)
    (g3,) = _make_mid(n, 256, 128, False)(
        p2a, p2b, dinv, b2.reshape(1, -1), W3)
    p3a, p3b = _make_prop_edge_split(n, ch3)(g3, zeros_tab, src2, dst2)
    out = _make_k4(n, 128)(p3a, p3b, dinv, b3.reshape(1, -1))
    return out


# speedup vs baseline: 23.9657x; 1.0101x over previous
import functools

import jax
import jax.numpy as jnp
from jax import lax
from jax.experimental import pallas as pl
from jax.experimental.pallas import tpu as pltpu
from jax.experimental.pallas import tpu_sc as plsc

F32 = jnp.float32
LANES = 128
NC = 2
NS = 16
DUMP = 32

_MESH = plsc.VectorSubcoreMesh(
    core_axis_name="c", subcore_axis_name="s", num_cores=NC, num_subcores=NS
)


IB = 8


def _run_edges(g_h, src_h, dst_h, ixs, ixd, jxs, jxd, r_a, r_b,
               s_a, s_b, s_i, acc, base_row, ch):
    nblk = ch // IB

    def pair(cs, cd, i0):
        i1 = i0 + 1
        pltpu.async_copy(g_h.at[cs.at[i1]], r_b, s_b)
        pltpu.make_async_copy(g_h.at[cs.at[i0]], r_a, s_a).wait()
        pltpu.sync_copy(r_a, acc.at[cd.at[i0]], add=True)
        pltpu.async_copy(g_h.at[cs.at[i0 + 2]], r_a, s_a)
        pltpu.make_async_copy(g_h.at[cs.at[i1]], r_b, s_b).wait()
        pltpu.sync_copy(r_b, acc.at[cd.at[i1]], add=True)

    def block(b, cs, cd, ns, nd):
        nb = jnp.where(b + 1 < nblk, b + 1, 0)
        row_n = base_row + nb * IB
        pltpu.async_copy(src_h.at[pl.ds(row_n, IB)], ns, s_i)
        pltpu.async_copy(dst_h.at[pl.ds(row_n, IB)], nd, s_i)

        def mid(j, c2):
            pair(cs, cd, 2 * j)
            return c2

        lax.fori_loop(0, IB // 2 - 1, mid, 0)
        pltpu.make_async_copy(src_h.at[pl.ds(row_n, IB)], ns, s_i).wait()
        pltpu.make_async_copy(dst_h.at[pl.ds(row_n, IB)], nd, s_i).wait()
        i0 = IB - 2
        i1 = IB - 1
        pltpu.async_copy(g_h.at[cs.at[i1]], r_b, s_b)
        pltpu.make_async_copy(g_h.at[cs.at[i0]], r_a, s_a).wait()
        pltpu.sync_copy(r_a, acc.at[cd.at[i0]], add=True)
        pltpu.async_copy(g_h.at[ns.at[0]], r_a, s_a)
        pltpu.make_async_copy(g_h.at[cs.at[i1]], r_b, s_b).wait()
        pltpu.sync_copy(r_b, acc.at[cd.at[i1]], add=True)

    pltpu.sync_copy(src_h.at[pl.ds(base_row, IB)], ixs)
    pltpu.sync_copy(dst_h.at[pl.ds(base_row, IB)], ixd)
    pltpu.async_copy(g_h.at[ixs.at[0]], r_a, s_a)

    def two(t, c2):
        block(2 * t, ixs, ixd, jxs, jxd)
        block(2 * t + 1, jxs, jxd, ixs, ixd)
        return c2

    lax.fori_loop(0, nblk // 2, two, 0)
    pltpu.make_async_copy(g_h.at[ixs.at[0]], r_a, s_a).wait()


def _tile_rows_copy(s, src_ref, dst_ref, rpt, last):
    @pl.when(s < NS - 1)
    def _():
        sl = pl.ds(s * rpt, rpt)
        pltpu.sync_copy(src_ref.at[sl], dst_ref.at[sl])

    @pl.when(s == NS - 1)
    def _():
        sl = pl.ds((NS - 1) * rpt, last)
        pltpu.sync_copy(src_ref.at[sl], dst_ref.at[sl])


@functools.lru_cache(maxsize=None)
def _make_prop_feature_split(n, ch):
    rpt = -(-n // (NS * 8)) * 8
    last = n - (NS - 1) * rpt
    out_t = [jax.ShapeDtypeStruct((n, 128), F32)] * 2
    scratch = [
        pltpu.VMEM((IB, LANES), jnp.int32),
        pltpu.VMEM((IB, LANES), jnp.int32),
        pltpu.VMEM((IB, LANES), jnp.int32),
        pltpu.VMEM((IB, LANES), jnp.int32),
        pltpu.VMEM((LANES, 128), F32),
        pltpu.VMEM((LANES, 128), F32),
        pltpu.SemaphoreType.DMA,
        pltpu.SemaphoreType.DMA,
        pltpu.SemaphoreType.DMA,
        pltpu.VMEM_SHARED((n + DUMP, 128), F32),
    ]

    @functools.partial(pl.kernel, out_type=out_t, mesh=_MESH,
                       scratch_types=scratch)
    def prop(ga_h, gb_h, src_h, dst_h, oa_h, ob_h,
             ixs, ixd, jxs, jxd, r_a, r_b, s_a, s_b, s_i, acc):
        c = lax.axis_index("c")
        s = lax.axis_index("s")

        @pl.when(c == 0)
        def _():
            _tile_rows_copy(s, ga_h, acc, rpt, last)

        @pl.when(c == 1)
        def _():
            _tile_rows_copy(s, gb_h, acc, rpt, last)

        plsc.subcore_barrier()

        @pl.when(c == 0)
        def _():
            _run_edges(ga_h, src_h, dst_h, ixs, ixd, jxs, jxd, r_a, r_b,
                       s_a, s_b, s_i, acc, s * ch, ch)

        @pl.when(c == 1)
        def _():
            _run_edges(gb_h, src_h, dst_h, ixs, ixd, jxs, jxd, r_a, r_b,
                       s_a, s_b, s_i, acc, s * ch, ch)

        plsc.subcore_barrier()

        @pl.when(c == 0)
        def _():
            _tile_rows_copy(s, acc, oa_h, rpt, last)

        @pl.when(c == 1)
        def _():
            _tile_rows_copy(s, acc, ob_h, rpt, last)

    return prop


@functools.lru_cache(maxsize=None)
def _make_prop_edge_split(n, ch):
    rpt = -(-n // (NS * 8)) * 8
    last = n - (NS - 1) * rpt
    out_t = [jax.ShapeDtypeStruct((n, 128), F32)] * 2
    scratch = [
        pltpu.VMEM((IB, LANES), jnp.int32),
        pltpu.VMEM((IB, LANES), jnp.int32),
        pltpu.VMEM((IB, LANES), jnp.int32),
        pltpu.VMEM((IB, LANES), jnp.int32),
        pltpu.VMEM((LANES, 128), F32),
        pltpu.VMEM((LANES, 128), F32),
        pltpu.SemaphoreType.DMA,
        pltpu.SemaphoreType.DMA,
        pltpu.SemaphoreType.DMA,
        pltpu.VMEM_SHARED((n + DUMP, 128), F32),
    ]

    @functools.partial(pl.kernel, out_type=out_t, mesh=_MESH,
                       scratch_types=scratch)
    def prop(g_h, z_h, src_h, dst_h, oa_h, ob_h,
             ixs, ixd, jxs, jxd, r_a, r_b, s_a, s_b, s_i, acc):
        c = lax.axis_index("c")
        s = lax.axis_index("s")
        wid = s * NC + c

        @pl.when(c == 0)
        def _():
            _tile_rows_copy(s, g_h, acc, rpt, last)

        @pl.when(c == 1)
        def _():
            _tile_rows_copy(s, z_h, acc, rpt, last)

        plsc.subcore_barrier()
        _run_edges(g_h, src_h, dst_h, ixs, ixd, jxs, jxd, r_a, r_b,
                   s_a, s_b, s_i, acc, wid * ch, ch)
        plsc.subcore_barrier()

        @pl.when(c == 0)
        def _():
            _tile_rows_copy(s, acc, oa_h, rpt, last)

        @pl.when(c == 1)
        def _():
            _tile_rows_copy(s, acc, ob_h, rpt, last)

    return prop


@functools.lru_cache(maxsize=None)
def _make_deg(n_acc, deg_tile, ch):
    out_t = [jax.ShapeDtypeStruct((n_acc,), F32)] * 2
    scratch = [
        pltpu.VMEM((ch, LANES), jnp.int32),
        pltpu.VMEM((LANES,), F32),
        pltpu.VMEM((deg_tile,), F32),
        pltpu.VMEM_SHARED((n_acc,), F32),
        pltpu.SemaphoreType.DMA,
    ]

    @functools.partial(pl.kernel, out_type=out_t, mesh=_MESH,
                       scratch_types=scratch)
    def deg(dst_h, oa_h, ob_h, idxbuf, ones_v, zbuf, dacc, sem):
        c = lax.axis_index("c")
        s = lax.axis_index("s")
        wid = s * NC + c

        def fill_z(i, carry):
            zbuf[pl.ds(i * 16, 16)] = jnp.zeros((16,), F32)
            return carry

        lax.fori_loop(0, deg_tile // 16, fill_z, 0)

        def fill_o(i, carry):
            ones_v[pl.ds(i * 16, 16)] = jnp.full((16,), 1.0, F32)
            return carry

        lax.fori_loop(0, LANES // 16, fill_o, 0)

        sl = pl.ds(s * deg_tile, deg_tile)
        pltpu.sync_copy(zbuf, dacc.at[sl])
        plsc.subcore_barrier()

        pltpu.sync_copy(dst_h.at[pl.ds(wid * ch, ch)], idxbuf)

        def st(i, carry):
            pltpu.async_copy(ones_v, dacc.at[idxbuf.at[i]], sem, add=True)
            return carry

        lax.fori_loop(0, ch, st, 0)

        def dr(i, carry):
            pltpu.make_async_copy(ones_v, dacc.at[idxbuf.at[i]], sem).wait()
            return carry

        lax.fori_loop(0, ch, dr, 0)
        plsc.subcore_barrier()

        @pl.when(c == 0)
        def _():
            pltpu.sync_copy(dacc.at[sl], oa_h.at[sl])

        @pl.when(c == 1)
        def _():
            pltpu.sync_copy(dacc.at[sl], ob_h.at[sl])

    return deg


_ROWS = 1000


def _row_spec(w):
    return pl.BlockSpec((_ROWS, w), lambda i: (i, 0))


def _full_spec(h, w):
    return pl.BlockSpec((h, w), lambda i: (0, 0))


@functools.lru_cache(maxsize=None)
def _make_k1(n, fin, fout):
    grid = (n // _ROWS,)

    def body(x_r, w_r, da_r, db_r, ga_r, gb_r, dv_r):
        dinv = lax.rsqrt(da_r[...] + db_r[...] + 1.0)
        g = jnp.dot(x_r[...], w_r[...], preferred_element_type=F32) * dinv
        ga_r[...] = g[:, : fout // 2]
        gb_r[...] = g[:, fout // 2:]
        dv_r[...] = dinv

    return pl.pallas_call(
        body,
        grid=grid,
        in_specs=[_row_spec(fin), _full_spec(fin, fout),
                  _row_spec(1), _row_spec(1)],
        out_specs=[_row_spec(fout // 2), _row_spec(fout // 2), _row_spec(1)],
        out_shape=[jax.ShapeDtypeStruct((n, fout // 2), F32),
                   jax.ShapeDtypeStruct((n, fout // 2), F32),
                   jax.ShapeDtypeStruct((n, 1), F32)],
    )


@functools.lru_cache(maxsize=None)
def _make_mid(n, fin, fout, split):
    grid = (n // _ROWS,)

    def body(pa_r, pb_r, dv_r, b_r, w_r, *outs):
        dinv = dv_r[...]
        p = jnp.concatenate([pa_r[...], pb_r[...]], axis=1)
        a = jnp.maximum(p * dinv + b_r[...], 0.0)
        g = jnp.dot(a, w_r[...], preferred_element_type=F32) * dinv
        if split:
            outs[0][...] = g[:, : fout // 2]
            outs[1][...] = g[:, fout // 2:]
        else:
            outs[0][...] = g

    if split:
        out_specs = [_row_spec(fout // 2), _row_spec(fout // 2)]
        out_shape = [jax.ShapeDtypeStruct((n, fout // 2), F32)] * 2
    else:
        out_specs = [_row_spec(fout)]
        out_shape = [jax.ShapeDtypeStruct((n, fout), F32)]

    return pl.pallas_call(
        body,
        grid=grid,
        in_specs=[_row_spec(fin // 2), _row_spec(fin // 2), _row_spec(1),
                  _full_spec(1, fin), _full_spec(fin, fout)],
        out_specs=out_specs,
        out_shape=out_shape,
    )


@functools.lru_cache(maxsize=None)
def _make_k4(n, f):
    grid = (n // _ROWS,)

    def body(pa_r, pb_r, dv_r, b_r, o_r):
        o_r[...] = jax.nn.sigmoid(
            (pa_r[...] + pb_r[...]) * dv_r[...] + b_r[...])

    return pl.pallas_call(
        body,
        grid=grid,
        in_specs=[_row_spec(f), _row_spec(f), _row_spec(1), _full_spec(1, f)],
        out_specs=_row_spec(f),
        out_shape=jax.ShapeDtypeStruct((n, f), F32),
    )


def kernel(x, edge_index, W1, b1, W2, b2, W3, b3):
    n = x.shape[0]
    e = edge_index.shape[1]

    src = edge_index[0].astype(jnp.int32)
    dst = edge_index[1].astype(jnp.int32)

    ch3 = -(-e // (NC * NS * LANES))
    ch3 += ch3 % 2
    ch1 = 2 * ch3
    e_pad = NC * NS * ch3 * LANES
    npad = e_pad - e
    ar = jnp.arange(npad, dtype=jnp.int32)
    src2 = jnp.concatenate([src, ar % n]).reshape(-1, LANES)
    dst2 = jnp.concatenate([dst, n + (ar % DUMP)]).reshape(-1, LANES)

    deg_tile = -(-(n + DUMP) // (NS * 16)) * 16
    n_acc = NS * deg_tile
    dega, degb = _make_deg(n_acc, deg_tile, ch3)(dst2)
    da = dega[:, None]
    db = degb[:, None]

    prop = _make_prop_feature_split(n, ch1)
    zeros_tab = jnp.zeros((n, 128), F32)

    g1a, g1b, dinv = _make_k1(n, 128, 256)(x, W1, da, db)
    p1a, p1b = prop(g1a, g1b, src2, dst2)
    g2a, g2b = _make_mid(n, 256, 256, True)(
        p1a, p1b, dinv, b1.reshape(1, -1), W2)
    p2a, p2b = prop(g2a, g2b, src2, dst2)
    (g3,) = _make_mid(n, 256, 128, False)(
        p2a, p2b, dinv, b2.reshape(1, -1), W3)
    p3a, p3b = _make_prop_edge_split(n, ch3)(g3, zeros_tab, src2, dst2)
    out = _make_k4(n, 128)(p3a, p3b, dinv, b3.reshape(1, -1))
    return out
